# baseline (speedup 1.0000x reference)
import functools
import math

import jax
import jax.numpy as jnp
from jax.experimental import pallas as pl
from jax.experimental.pallas import tpu as pltpu
from jax.experimental.pallas import tpu_sc as plsc

_NOPE = 128
_ROPE = 64
_VHD = 128
_EPS = 1e-6


def _rms(x, w):
    var = jnp.mean(x * x, axis=-1, keepdims=True)
    return (x * jax.lax.rsqrt(var + _EPS)) * w


def _rot_half(x):
    half = x.shape[-1] // 2
    return jnp.concatenate([-x[:, half:], x[:, :half]], axis=-1)


def _nt_dot(a, b):
    return jax.lax.dot_general(a, b, (((1,), (1,)), ((), ())),
                               preferred_element_type=jnp.float32)


def _k1_body(nh, nope, rope, hid_ref, cos_ref, sin_ref, ln1_ref, wqa_ref, qnw_ref,
             wqb_ref, wkva_ref, kvnw_ref, wkvbn_ref,
             q2_ref, qpe_ref, kv_ref, kpe_ref):
    x = hid_ref[...]
    xb = _rms(x, ln1_ref[...]).astype(jnp.bfloat16)
    qa = jnp.dot(xb, wqa_ref[...], preferred_element_type=jnp.float32)
    qab = _rms(qa, qnw_ref[...]).astype(jnp.bfloat16)
    q = jnp.dot(qab, wqb_ref[...], preferred_element_type=jnp.float32)
    kvf = jnp.dot(xb, wkva_ref[...], preferred_element_type=jnp.float32)
    kvlr = kvnw_ref.shape[-1]
    kv_ref[...] = _rms(kvf[:, :kvlr], kvnw_ref[...]).astype(jnp.bfloat16)
    cos = cos_ref[...]
    sin = sin_ref[...]
    kpe = kvf[:, kvlr:]
    kpe_ref[...] = (kpe * cos + _rot_half(kpe) * sin).astype(jnp.bfloat16)
    qkhd = nope + rope
    for h in range(nh):
        qn = q[:, h * qkhd:h * qkhd + nope].astype(jnp.bfloat16)
        q2_ref[h] = jnp.dot(qn, wkvbn_ref[h],
                            preferred_element_type=jnp.float32).astype(jnp.bfloat16)
        qp = q[:, h * qkhd + nope:(h + 1) * qkhd]
        qpe_ref[h] = (qp * cos + _rot_half(qp) * sin).astype(jnp.bfloat16)


def _k2_body(scale, nb, q2_ref, qpe_ref, kv_ref, kpe_ref, o_ref,
             m_ref, l_ref, acc_ref):
    qb = pl.program_id(1)
    kvb = pl.program_id(2)
    bt = q2_ref.shape[1]
    kvlr = kv_ref.shape[-1]

    @pl.when(kvb == 0)
    def _():
        m_ref[...] = jnp.full_like(m_ref, -1e30)
        l_ref[...] = jnp.zeros_like(l_ref)
        acc_ref[...] = jnp.zeros_like(acc_ref)

    @pl.when(kvb <= qb)
    def _():
        q2 = q2_ref[0]
        qpe = qpe_ref[0]
        kc = kv_ref[pl.ds(kvb * bt, bt), :]
        pc = kpe_ref[pl.ds(kvb * bt, bt), :]
        s = _nt_dot(q2, kc) + _nt_dot(qpe, pc)
        s = s * scale

        @pl.when(kvb == qb)
        def _():
            rows = jax.lax.broadcasted_iota(jnp.int32, (bt, bt), 0)
            cols = jax.lax.broadcasted_iota(jnp.int32, (bt, bt), 1)
            sm = jnp.where(cols > rows, -1e30, s)
            _k2_update(sm, kc, m_ref, l_ref, acc_ref)

        @pl.when(kvb < qb)
        def _():
            _k2_update(s, kc, m_ref, l_ref, acc_ref)

    @pl.when(kvb == nb - 1)
    def _():
        o_ref[0] = (acc_ref[...] / l_ref[...]).astype(jnp.bfloat16)


def _k2_update(s, kc, m_ref, l_ref, acc_ref):
    m = m_ref[...]
    mnew = jnp.maximum(m, jnp.max(s, axis=-1, keepdims=True))
    p = jnp.exp(s - mnew)
    alpha = jnp.exp(m - mnew)
    m_ref[...] = mnew
    l_ref[...] = l_ref[...] * alpha + jnp.sum(p, axis=-1, keepdims=True)
    acc_ref[...] = acc_ref[...] * alpha + jnp.dot(
        p.astype(jnp.bfloat16), kc, preferred_element_type=jnp.float32)


def _k3_body(nh, o_ref, wv_ref, woT_ref, wob_ref, hid_ref, ln2_ref, gT_ref, gb_ref,
             h2_ref, y_ref, lg_ref, i1_ref, i2_ref, w1_ref, w2_ref):
    parts = [jnp.dot(o_ref[h], wv_ref[h], preferred_element_type=jnp.float32)
             for h in range(nh)]
    o2 = jnp.concatenate(parts, axis=-1).astype(jnp.bfloat16)
    attn = jnp.dot(o2, woT_ref[...], preferred_element_type=jnp.float32) + wob_ref[...]
    h2 = hid_ref[...] + attn
    h2_ref[...] = h2
    y = _rms(h2, ln2_ref[...])
    y_ref[...] = y
    yb = y.astype(jnp.bfloat16)
    lg = jnp.dot(yb, gT_ref[...], preferred_element_type=jnp.float32) + gb_ref[...]
    lg_ref[...] = lg
    en = lg.shape[-1]
    col = jax.lax.broadcasted_iota(jnp.int32, lg.shape, 1)
    m1 = jnp.max(lg, axis=-1, keepdims=True)
    i1 = jnp.min(jnp.where(lg == m1, col, en), axis=-1, keepdims=True)
    l2 = jnp.where(col == i1, -jnp.inf, lg)
    m2 = jnp.max(l2, axis=-1, keepdims=True)
    i2 = jnp.min(jnp.where(l2 == m2, col, en), axis=-1, keepdims=True)
    i1_ref[...] = i1
    i2_ref[...] = i2
    e2 = jnp.exp(m2 - m1)
    denom = 1.0 + e2
    w1_ref[...] = 1.0 / denom
    w2_ref[...] = e2 / denom


def _k4_body(en, blk, npb, i1_ref, i2_ref, pos1_ref, pos2_ref, idx_ref, be_ref):
    i1 = i1_ref[...]
    i2 = i2_ref[...]
    s = i1.shape[0]
    eio = jax.lax.broadcasted_iota(jnp.int32, (s, en), 1)
    m1 = (i1 == eio).astype(jnp.float32)
    m2 = (i2 == eio).astype(jnp.float32)
    cnt1 = jnp.sum(m1, axis=0, keepdims=True)
    cnt2 = jnp.sum(m2, axis=0, keepdims=True)
    cnt = cnt1 + cnt2
    pc = jnp.floor((cnt + (blk - 1)) / blk) * blk
    r8 = jax.lax.broadcasted_iota(jnp.int32, (en, en), 0)
    c8 = jax.lax.broadcasted_iota(jnp.int32, (en, en), 1)
    tri = (r8 < c8).astype(jnp.float32)
    off = jnp.dot(pc, tri, preferred_element_type=jnp.float32)
    nc = s // blk
    p1c, p2c = [], []
    for c in range(nc):
        rowg = c * blk + jax.lax.broadcasted_iota(jnp.int32, (blk, s), 0)
        colg = jax.lax.broadcasted_iota(jnp.int32, (blk, s), 1)
        lt = (colg < rowg).astype(jnp.float32)
        p1c.append(jnp.dot(lt, m1, preferred_element_type=jnp.float32))
        p2c.append(jnp.dot(lt, m2, preferred_element_type=jnp.float32))
    pref1 = jnp.concatenate(p1c, axis=0)
    pref2 = jnp.concatenate(p2c, axis=0)
    rank1 = jnp.sum(pref1 * m1, axis=1, keepdims=True)
    rank2 = jnp.sum(pref2 * m2, axis=1, keepdims=True)
    off1 = jnp.sum(off * m1, axis=1, keepdims=True)
    base2 = off + cnt1
    off2 = jnp.sum(base2 * m2, axis=1, keepdims=True)
    pos1 = off1 + rank1
    pos2 = off2 + rank2
    pos1_ref[...] = pos1.astype(jnp.int32)
    pos2_ref[...] = pos2.astype(jnp.int32)
    tok = jax.lax.broadcasted_iota(jnp.int32, (s, blk), 0)
    p1i = pos1.astype(jnp.int32)
    p2i = pos2.astype(jnp.int32)
    rows = []
    for c in range(npb):
        slot = c * blk + jax.lax.broadcasted_iota(jnp.int32, (s, blk), 1)
        hit1 = jnp.where(p1i == slot, tok, -1)
        hit2 = jnp.where(p2i == slot, tok, -1)
        row = jnp.maximum(jnp.max(hit1, axis=0, keepdims=True),
                          jnp.max(hit2, axis=0, keepdims=True))
        rows.append(row)
    idx_ref[...] = jnp.maximum(jnp.concatenate(rows, axis=0), 0)
    bstart = jax.lax.broadcasted_iota(jnp.int32, (npb, en), 0) * blk
    offi = off.astype(jnp.int32)
    be = jnp.sum((bstart >= offi).astype(jnp.int32), axis=1, keepdims=True) - 1
    be_ref[...] = jnp.clip(be, 0, en - 1)


def _k6_body(be_ref, ys_ref, eg_ref, eu_ref, edT_ref, eo_ref):
    x = ys_ref[...].astype(jnp.bfloat16)
    g = jnp.dot(x, eg_ref[0], preferred_element_type=jnp.float32)
    u = jnp.dot(x, eu_ref[0], preferred_element_type=jnp.float32)
    act = (g * jax.nn.sigmoid(g) * u).astype(jnp.bfloat16)
    eo_ref[...] = jnp.dot(act, edT_ref[0], preferred_element_type=jnp.float32)


def _k7_body(g1_ref, g2_ref, h2_ref, w1_ref, w2_ref, out_ref):
    out_ref[...] = (h2_ref[...]
                    + w1_ref[...] * g1_ref[...]
                    + w2_ref[...] * g2_ref[...])


def _sc_gather(table, idx):
    info = plsc.get_sparse_core_info()
    nw = info.num_cores * info.num_subcores
    b = idx.shape[0]
    d = table.shape[1]
    b_per_w = b // nw
    ch = 16 if b_per_w % 16 == 0 else 8
    n_ch = b_per_w // ch
    nc = info.num_cores
    mesh = plsc.VectorSubcoreMesh(core_axis_name="c", subcore_axis_name="s")

    @functools.partial(
        pl.kernel, mesh=mesh,
        out_type=jax.ShapeDtypeStruct((b, d), table.dtype),
        scratch_types=[
            pltpu.VMEM((ch,), jnp.int32),
            pltpu.VMEM((ch, d), table.dtype),
            pltpu.SemaphoreType.DMA,
        ],
    )
    def k(table_hbm, idx_hbm, out_hbm, idx_v, rows_v, sem):
        wid = jax.lax.axis_index("s") * nc + jax.lax.axis_index("c")
        base = wid * b_per_w

        def step(j, carry):
            off = base + j * ch
            pltpu.sync_copy(idx_hbm.at[pl.ds(off, ch)], idx_v)
            pltpu.async_copy(table_hbm.at[idx_v], rows_v, sem).wait()
            pltpu.sync_copy(rows_v, out_hbm.at[pl.ds(off, ch)])
            return carry

        jax.lax.fori_loop(0, n_ch, step, 0)

    return k(table, idx)


def kernel(hidden_state, attention_mask, ln1_w, ln2_w, wq_a_w, wq_a_b, q_norm_w,
           wq_b_w, wq_b_b, wkv_a_w, wkv_a_b, kv_norm_w, wkv_b_w, wo_w, wo_b,
           gate_w, gate_b, eg_w, eu_w, ed_w):
    bs, S, HS = hidden_state.shape
    hid = hidden_state.reshape(S, HS)
    QLR = wq_a_w.shape[0]
    NH = wq_b_w.shape[0] // (_NOPE + _ROPE)
    KVLR = kv_norm_w.shape[0]
    EN, EK, _ = eg_w.shape
    qkhd = _NOPE + _ROPE
    scale = 1.0 / math.sqrt(float(qkhd))

    inv_freq = 1.0 / (10000.0 ** (jnp.arange(0, _ROPE, 2, dtype=jnp.float32) / _ROPE))
    t = jnp.arange(S, dtype=jnp.float32)[:, None]
    freqs = t * inv_freq[None, :]
    freqs = jnp.concatenate([freqs, freqs], axis=-1)
    cos = jnp.cos(freqs)
    sin = jnp.sin(freqs)

    f16 = jnp.bfloat16
    wqaT = wq_a_w.T.astype(f16)
    wqbT = wq_b_w.T.astype(f16)
    wkvaT = wkv_a_w.T.astype(f16)
    wkvb = wkv_b_w.reshape(NH, _NOPE + _VHD, KVLR)
    wkvbn = wkvb[:, :_NOPE, :].astype(f16)
    wv = wkvb[:, _NOPE:, :].transpose(0, 2, 1).astype(f16)
    woT = wo_w.T.astype(f16)
    gT = gate_w.T.astype(f16)
    egT = eg_w.transpose(0, 2, 1).astype(f16)
    euT = eu_w.transpose(0, 2, 1).astype(f16)
    edT = ed_w.transpose(0, 2, 1).astype(f16)
    ln1 = ln1_w.reshape(1, HS)
    ln2 = ln2_w.reshape(1, HS)
    qnw = q_norm_w.reshape(1, QLR)
    kvnw = kv_norm_w.reshape(1, KVLR)
    wob = wo_b.reshape(1, HS)
    gb = gate_b.reshape(1, EN)

    BT = min(256, S)
    NB = S // BT

    k1 = pl.pallas_call(
        functools.partial(_k1_body, NH, _NOPE, _ROPE),
        grid=(NB,),
        in_specs=[
            pl.BlockSpec((BT, HS), lambda i: (i, 0)),
            pl.BlockSpec((BT, _ROPE), lambda i: (i, 0)),
            pl.BlockSpec((BT, _ROPE), lambda i: (i, 0)),
            pl.BlockSpec((1, HS), lambda i: (0, 0)),
            pl.BlockSpec((HS, QLR), lambda i: (0, 0)),
            pl.BlockSpec((1, QLR), lambda i: (0, 0)),
            pl.BlockSpec((QLR, NH * qkhd), lambda i: (0, 0)),
            pl.BlockSpec((HS, KVLR + _ROPE), lambda i: (0, 0)),
            pl.BlockSpec((1, KVLR), lambda i: (0, 0)),
            pl.BlockSpec((NH, _NOPE, KVLR), lambda i: (0, 0, 0)),
        ],
        out_specs=[
            pl.BlockSpec((NH, BT, KVLR), lambda i: (0, i, 0)),
            pl.BlockSpec((NH, BT, _ROPE), lambda i: (0, i, 0)),
            pl.BlockSpec((BT, KVLR), lambda i: (i, 0)),
            pl.BlockSpec((BT, _ROPE), lambda i: (i, 0)),
        ],
        out_shape=[
            jax.ShapeDtypeStruct((NH, S, KVLR), f16),
            jax.ShapeDtypeStruct((NH, S, _ROPE), f16),
            jax.ShapeDtypeStruct((S, KVLR), f16),
            jax.ShapeDtypeStruct((S, _ROPE), f16),
        ],
        compiler_params=pltpu.CompilerParams(
            dimension_semantics=("arbitrary",)),
    )
    q2, qpe, kv, kpe = k1(hid, cos, sin, ln1, wqaT, qnw, wqbT, wkvaT, kvnw, wkvbn)

    k2 = pl.pallas_call(
        functools.partial(_k2_body, scale, NB),
        grid=(NH, NB, NB),
        in_specs=[
            pl.BlockSpec((1, BT, KVLR), lambda h, i, j: (h, i, 0)),
            pl.BlockSpec((1, BT, _ROPE), lambda h, i, j: (h, i, 0)),
            pl.BlockSpec((S, KVLR), lambda h, i, j: (0, 0)),
            pl.BlockSpec((S, _ROPE), lambda h, i, j: (0, 0)),
        ],
        out_specs=pl.BlockSpec((1, BT, KVLR), lambda h, i, j: (h, i, 0)),
        out_shape=jax.ShapeDtypeStruct((NH, S, KVLR), f16),
        scratch_shapes=[
            pltpu.VMEM((BT, 1), jnp.float32),
            pltpu.VMEM((BT, 1), jnp.float32),
            pltpu.VMEM((BT, KVLR), jnp.float32),
        ],
        compiler_params=pltpu.CompilerParams(
            dimension_semantics=("arbitrary", "arbitrary", "arbitrary")),
    )
    o = k2(q2, qpe, kv, kpe)

    k3 = pl.pallas_call(
        functools.partial(_k3_body, NH),
        grid=(NB,),
        in_specs=[
            pl.BlockSpec((NH, BT, KVLR), lambda i: (0, i, 0)),
            pl.BlockSpec((NH, KVLR, _VHD), lambda i: (0, 0, 0)),
            pl.BlockSpec((NH * _VHD, HS), lambda i: (0, 0)),
            pl.BlockSpec((1, HS), lambda i: (0, 0)),
            pl.BlockSpec((BT, HS), lambda i: (i, 0)),
            pl.BlockSpec((1, HS), lambda i: (0, 0)),
            pl.BlockSpec((HS, EN), lambda i: (0, 0)),
            pl.BlockSpec((1, EN), lambda i: (0, 0)),
        ],
        out_specs=[
            pl.BlockSpec((BT, HS), lambda i: (i, 0)),
            pl.BlockSpec((BT, HS), lambda i: (i, 0)),
            pl.BlockSpec((BT, EN), lambda i: (i, 0)),
            pl.BlockSpec((BT, 1), lambda i: (i, 0)),
            pl.BlockSpec((BT, 1), lambda i: (i, 0)),
            pl.BlockSpec((BT, 1), lambda i: (i, 0)),
            pl.BlockSpec((BT, 1), lambda i: (i, 0)),
        ],
        out_shape=[
            jax.ShapeDtypeStruct((S, HS), jnp.float32),
            jax.ShapeDtypeStruct((S, HS), jnp.float32),
            jax.ShapeDtypeStruct((S, EN), jnp.float32),
            jax.ShapeDtypeStruct((S, 1), jnp.int32),
            jax.ShapeDtypeStruct((S, 1), jnp.int32),
            jax.ShapeDtypeStruct((S, 1), jnp.float32),
            jax.ShapeDtypeStruct((S, 1), jnp.float32),
        ],
        compiler_params=pltpu.CompilerParams(
            dimension_semantics=("arbitrary",)),
    )
    h2, y, logits, i1, i2, w1, w2 = k3(o, wv, woT, wob, hid, ln2, gT, gb)

    BLKM = min(256, S)
    NPB = (2 * S) // BLKM + EN
    k4 = pl.pallas_call(
        functools.partial(_k4_body, EN, BLKM, NPB),
        grid=(1,),
        in_specs=[
            pl.BlockSpec((S, 1), lambda i: (0, 0)),
            pl.BlockSpec((S, 1), lambda i: (0, 0)),
        ],
        out_specs=[
            pl.BlockSpec((S, 1), lambda i: (0, 0)),
            pl.BlockSpec((S, 1), lambda i: (0, 0)),
            pl.BlockSpec((NPB, BLKM), lambda i: (0, 0)),
            pl.BlockSpec((NPB, 1), lambda i: (0, 0)),
        ],
        out_shape=[
            jax.ShapeDtypeStruct((S, 1), jnp.int32),
            jax.ShapeDtypeStruct((S, 1), jnp.int32),
            jax.ShapeDtypeStruct((NPB, BLKM), jnp.int32),
            jax.ShapeDtypeStruct((NPB, 1), jnp.int32),
        ],
    )
    pos1, pos2, idx_sorted, blk_exp = k4(i1, i2)

    idx_flat = idx_sorted.reshape(NPB * BLKM)
    be_flat = blk_exp.reshape(NPB)
    pos_flat = jnp.concatenate([pos1.reshape(S), pos2.reshape(S)])

    ys = _sc_gather(y, idx_flat)

    k6 = pl.pallas_call(
        _k6_body,
        grid_spec=pltpu.PrefetchScalarGridSpec(
            num_scalar_prefetch=1,
            grid=(NPB,),
            in_specs=[
                pl.BlockSpec((BLKM, HS), lambda b, bes: (b, 0)),
                pl.BlockSpec((1, HS, EK), lambda b, bes: (bes[b], 0, 0)),
                pl.BlockSpec((1, HS, EK), lambda b, bes: (bes[b], 0, 0)),
                pl.BlockSpec((1, EK, HS), lambda b, bes: (bes[b], 0, 0)),
            ],
            out_specs=pl.BlockSpec((BLKM, HS), lambda b, bes: (b, 0)),
        ),
        out_shape=jax.ShapeDtypeStruct((NPB * BLKM, HS), jnp.float32),
        compiler_params=pltpu.CompilerParams(
            dimension_semantics=("arbitrary",)),
    )
    eo = k6(be_flat, ys, egT, euT, edT)

    go = _sc_gather(eo, pos_flat)

    k7 = pl.pallas_call(
        _k7_body,
        grid=(NB,),
        in_specs=[
            pl.BlockSpec((BT, HS), lambda i: (i, 0)),
            pl.BlockSpec((BT, HS), lambda i: (NB + i, 0)),
            pl.BlockSpec((BT, HS), lambda i: (i, 0)),
            pl.BlockSpec((BT, 1), lambda i: (i, 0)),
            pl.BlockSpec((BT, 1), lambda i: (i, 0)),
        ],
        out_specs=pl.BlockSpec((BT, HS), lambda i: (i, 0)),
        out_shape=jax.ShapeDtypeStruct((S, HS), jnp.float32),
        compiler_params=pltpu.CompilerParams(
            dimension_semantics=("arbitrary",)),
    )
    out = k7(go, go, h2, w1, w2)

    return out.reshape(bs, S, HS), logits


# speedup vs baseline: 1.4348x; 1.4348x over previous
import functools
import math

import jax
import jax.numpy as jnp
from jax.experimental import pallas as pl
from jax.experimental.pallas import tpu as pltpu
from jax.experimental.pallas import tpu_sc as plsc

_NOPE = 128
_ROPE = 64
_VHD = 128
_EPS = 1e-6


def _rms(x, w):
    var = jnp.mean(x * x, axis=-1, keepdims=True)
    return (x * jax.lax.rsqrt(var + _EPS)) * w


def _rot_half(x):
    half = x.shape[-1] // 2
    return jnp.concatenate([-x[:, half:], x[:, :half]], axis=-1)


def _nt_dot(a, b):
    return jax.lax.dot_general(a, b, (((1,), (1,)), ((), ())),
                               preferred_element_type=jnp.float32)


def _k1_body(nh, nope, rope, hid_ref, cos_ref, sin_ref, ln1_ref, wqa_ref, qnw_ref,
             wqb_ref, wkva_ref, kvnw_ref, wkvbn_ref,
             q2_ref, qpe_ref, kv_ref, kpe_ref):
    x = hid_ref[...]
    xb = _rms(x, ln1_ref[...]).astype(jnp.bfloat16)
    qa = jnp.dot(xb, wqa_ref[...], preferred_element_type=jnp.float32)
    qab = _rms(qa, qnw_ref[...]).astype(jnp.bfloat16)
    q = jnp.dot(qab, wqb_ref[...], preferred_element_type=jnp.float32)
    kvf = jnp.dot(xb, wkva_ref[...], preferred_element_type=jnp.float32)
    kvlr = kvnw_ref.shape[-1]
    kv_ref[...] = _rms(kvf[:, :kvlr], kvnw_ref[...]).astype(jnp.bfloat16)
    cos = cos_ref[...]
    sin = sin_ref[...]
    kpe = kvf[:, kvlr:]
    kpe_ref[...] = (kpe * cos + _rot_half(kpe) * sin).astype(jnp.bfloat16)
    qkhd = nope + rope
    for h in range(nh):
        qn = q[:, h * qkhd:h * qkhd + nope].astype(jnp.bfloat16)
        q2_ref[h] = jnp.dot(qn, wkvbn_ref[h],
                            preferred_element_type=jnp.float32).astype(jnp.bfloat16)
        qp = q[:, h * qkhd + nope:(h + 1) * qkhd]
        qpe_ref[h] = (qp * cos + _rot_half(qp) * sin).astype(jnp.bfloat16)


def _k2_body(scale, q2_ref, qpe_ref, kv_ref, kpe_ref, o_ref):
    qb = pl.program_id(1)
    bt = q2_ref.shape[1]
    s_all = kv_ref.shape[0]
    q2 = q2_ref[0]
    qpe = qpe_ref[0]
    s = _nt_dot(q2, kv_ref[...]) + _nt_dot(qpe, kpe_ref[...])
    s = s * scale
    rows = qb * bt + jax.lax.broadcasted_iota(jnp.int32, (bt, s_all), 0)
    cols = jax.lax.broadcasted_iota(jnp.int32, (bt, s_all), 1)
    s = jnp.where(cols > rows, -1e30, s)
    m = jnp.max(s, axis=-1, keepdims=True)
    p = jnp.exp(s - m)
    l = jnp.sum(p, axis=-1, keepdims=True)
    acc = jnp.dot(p.astype(jnp.bfloat16), kv_ref[...],
                  preferred_element_type=jnp.float32)
    o_ref[0] = (acc / l).astype(jnp.bfloat16)


def _k3_body(nh, o_ref, wv_ref, woT_ref, wob_ref, hid_ref, ln2_ref, gT_ref, gb_ref,
             h2_ref, y_ref, lg_ref, i1_ref, i2_ref, w1_ref, w2_ref):
    parts = [jnp.dot(o_ref[h], wv_ref[h], preferred_element_type=jnp.float32)
             for h in range(nh)]
    o2 = jnp.concatenate(parts, axis=-1).astype(jnp.bfloat16)
    attn = jnp.dot(o2, woT_ref[...], preferred_element_type=jnp.float32) + wob_ref[...]
    h2 = hid_ref[...] + attn
    h2_ref[...] = h2
    y = _rms(h2, ln2_ref[...])
    y_ref[...] = y
    yb = y.astype(jnp.bfloat16)
    lg = jnp.dot(yb, gT_ref[...], preferred_element_type=jnp.float32) + gb_ref[...]
    lg_ref[...] = lg
    en = lg.shape[-1]
    col = jax.lax.broadcasted_iota(jnp.int32, lg.shape, 1)
    m1 = jnp.max(lg, axis=-1, keepdims=True)
    i1 = jnp.min(jnp.where(lg == m1, col, en), axis=-1, keepdims=True)
    l2 = jnp.where(col == i1, -jnp.inf, lg)
    m2 = jnp.max(l2, axis=-1, keepdims=True)
    i2 = jnp.min(jnp.where(l2 == m2, col, en), axis=-1, keepdims=True)
    i1_ref[...] = i1
    i2_ref[...] = i2
    e2 = jnp.exp(m2 - m1)
    denom = 1.0 + e2
    w1_ref[...] = 1.0 / denom
    w2_ref[...] = e2 / denom


def _k4_body(en, blk, npb, i1_ref, i2_ref, pos1_ref, pos2_ref, idx_ref, be_ref):
    i1 = i1_ref[...]
    i2 = i2_ref[...]
    s = i1.shape[0]
    eio = jax.lax.broadcasted_iota(jnp.int32, (s, en), 1)
    m1 = (i1 == eio).astype(jnp.float32)
    m2 = (i2 == eio).astype(jnp.float32)
    cnt1 = jnp.sum(m1, axis=0, keepdims=True)
    cnt2 = jnp.sum(m2, axis=0, keepdims=True)
    cnt = cnt1 + cnt2
    pc = jnp.floor((cnt + (blk - 1)) / blk) * blk
    r8 = jax.lax.broadcasted_iota(jnp.int32, (en, en), 0)
    c8 = jax.lax.broadcasted_iota(jnp.int32, (en, en), 1)
    tri = (r8 < c8).astype(jnp.float32)
    off = jnp.dot(pc, tri, preferred_element_type=jnp.float32)
    nc = s // blk
    p1c, p2c = [], []
    for c in range(nc):
        rowg = c * blk + jax.lax.broadcasted_iota(jnp.int32, (blk, s), 0)
        colg = jax.lax.broadcasted_iota(jnp.int32, (blk, s), 1)
        lt = (colg < rowg).astype(jnp.float32)
        p1c.append(jnp.dot(lt, m1, preferred_element_type=jnp.float32))
        p2c.append(jnp.dot(lt, m2, preferred_element_type=jnp.float32))
    pref1 = jnp.concatenate(p1c, axis=0)
    pref2 = jnp.concatenate(p2c, axis=0)
    rank1 = jnp.sum(pref1 * m1, axis=1, keepdims=True)
    rank2 = jnp.sum(pref2 * m2, axis=1, keepdims=True)
    off1 = jnp.sum(off * m1, axis=1, keepdims=True)
    base2 = off + cnt1
    off2 = jnp.sum(base2 * m2, axis=1, keepdims=True)
    pos1 = off1 + rank1
    pos2 = off2 + rank2
    pos1_ref[...] = pos1.astype(jnp.int32)
    pos2_ref[...] = pos2.astype(jnp.int32)
    tok = jax.lax.broadcasted_iota(jnp.int32, (s, blk), 0)
    p1i = pos1.astype(jnp.int32)
    p2i = pos2.astype(jnp.int32)
    rows = []
    for c in range(npb):
        slot = c * blk + jax.lax.broadcasted_iota(jnp.int32, (s, blk), 1)
        hit1 = jnp.where(p1i == slot, tok, -1)
        hit2 = jnp.where(p2i == slot, tok, -1)
        row = jnp.maximum(jnp.max(hit1, axis=0, keepdims=True),
                          jnp.max(hit2, axis=0, keepdims=True))
        rows.append(row)
    idx_ref[...] = jnp.maximum(jnp.concatenate(rows, axis=0), 0)
    bstart = jax.lax.broadcasted_iota(jnp.int32, (npb, en), 0) * blk
    offi = off.astype(jnp.int32)
    be = jnp.sum((bstart >= offi).astype(jnp.int32), axis=1, keepdims=True) - 1
    be_ref[...] = jnp.clip(be, 0, en - 1)


def _k6_body(be_ref, ys_ref, eg_ref, eu_ref, edT_ref, eo_ref):
    x = ys_ref[...].astype(jnp.bfloat16)
    g = jnp.dot(x, eg_ref[0], preferred_element_type=jnp.float32)
    u = jnp.dot(x, eu_ref[0], preferred_element_type=jnp.float32)
    act = (g * jax.nn.sigmoid(g) * u).astype(jnp.bfloat16)
    eo_ref[...] = jnp.dot(act, edT_ref[0], preferred_element_type=jnp.float32)


def _k7_body(g1_ref, g2_ref, h2_ref, w1_ref, w2_ref, out_ref):
    out_ref[...] = (h2_ref[...]
                    + w1_ref[...] * g1_ref[...]
                    + w2_ref[...] * g2_ref[...])


def _sc_gather(table, idx):
    info = plsc.get_sparse_core_info()
    nw = info.num_cores * info.num_subcores
    b = idx.shape[0]
    d = table.shape[1]
    b_per_w = b // nw
    ch = 16 if b_per_w % 16 == 0 else 8
    n_ch = b_per_w // ch
    nc = info.num_cores
    mesh = plsc.VectorSubcoreMesh(core_axis_name="c", subcore_axis_name="s")

    @functools.partial(
        pl.kernel, mesh=mesh,
        out_type=jax.ShapeDtypeStruct((b, d), table.dtype),
        scratch_types=[
            pltpu.VMEM((ch,), jnp.int32),
            pltpu.VMEM((ch, d), table.dtype),
            pltpu.SemaphoreType.DMA,
        ],
    )
    def k(table_hbm, idx_hbm, out_hbm, idx_v, rows_v, sem):
        wid = jax.lax.axis_index("s") * nc + jax.lax.axis_index("c")
        base = wid * b_per_w

        def step(j, carry):
            off = base + j * ch
            pltpu.sync_copy(idx_hbm.at[pl.ds(off, ch)], idx_v)
            pltpu.async_copy(table_hbm.at[idx_v], rows_v, sem).wait()
            pltpu.sync_copy(rows_v, out_hbm.at[pl.ds(off, ch)])
            return carry

        jax.lax.fori_loop(0, n_ch, step, 0)

    return k(table, idx)


def kernel(hidden_state, attention_mask, ln1_w, ln2_w, wq_a_w, wq_a_b, q_norm_w,
           wq_b_w, wq_b_b, wkv_a_w, wkv_a_b, kv_norm_w, wkv_b_w, wo_w, wo_b,
           gate_w, gate_b, eg_w, eu_w, ed_w):
    bs, S, HS = hidden_state.shape
    hid = hidden_state.reshape(S, HS)
    QLR = wq_a_w.shape[0]
    NH = wq_b_w.shape[0] // (_NOPE + _ROPE)
    KVLR = kv_norm_w.shape[0]
    EN, EK, _ = eg_w.shape
    qkhd = _NOPE + _ROPE
    scale = 1.0 / math.sqrt(float(qkhd))

    inv_freq = 1.0 / (10000.0 ** (jnp.arange(0, _ROPE, 2, dtype=jnp.float32) / _ROPE))
    t = jnp.arange(S, dtype=jnp.float32)[:, None]
    freqs = t * inv_freq[None, :]
    freqs = jnp.concatenate([freqs, freqs], axis=-1)
    cos = jnp.cos(freqs)
    sin = jnp.sin(freqs)

    f16 = jnp.bfloat16
    wqaT = wq_a_w.T.astype(f16)
    wqbT = wq_b_w.T.astype(f16)
    wkvaT = wkv_a_w.T.astype(f16)
    wkvb = wkv_b_w.reshape(NH, _NOPE + _VHD, KVLR)
    wkvbn = wkvb[:, :_NOPE, :].astype(f16)
    wv = wkvb[:, _NOPE:, :].transpose(0, 2, 1).astype(f16)
    woT = wo_w.T.astype(f16)
    gT = gate_w.T.astype(f16)
    egT = eg_w.transpose(0, 2, 1).astype(f16)
    euT = eu_w.transpose(0, 2, 1).astype(f16)
    edT = ed_w.transpose(0, 2, 1).astype(f16)
    ln1 = ln1_w.reshape(1, HS)
    ln2 = ln2_w.reshape(1, HS)
    qnw = q_norm_w.reshape(1, QLR)
    kvnw = kv_norm_w.reshape(1, KVLR)
    wob = wo_b.reshape(1, HS)
    gb = gate_b.reshape(1, EN)

    BT = min(256, S)
    NB = S // BT

    k1 = pl.pallas_call(
        functools.partial(_k1_body, NH, _NOPE, _ROPE),
        grid=(NB,),
        in_specs=[
            pl.BlockSpec((BT, HS), lambda i: (i, 0)),
            pl.BlockSpec((BT, _ROPE), lambda i: (i, 0)),
            pl.BlockSpec((BT, _ROPE), lambda i: (i, 0)),
            pl.BlockSpec((1, HS), lambda i: (0, 0)),
            pl.BlockSpec((HS, QLR), lambda i: (0, 0)),
            pl.BlockSpec((1, QLR), lambda i: (0, 0)),
            pl.BlockSpec((QLR, NH * qkhd), lambda i: (0, 0)),
            pl.BlockSpec((HS, KVLR + _ROPE), lambda i: (0, 0)),
            pl.BlockSpec((1, KVLR), lambda i: (0, 0)),
            pl.BlockSpec((NH, _NOPE, KVLR), lambda i: (0, 0, 0)),
        ],
        out_specs=[
            pl.BlockSpec((NH, BT, KVLR), lambda i: (0, i, 0)),
            pl.BlockSpec((NH, BT, _ROPE), lambda i: (0, i, 0)),
            pl.BlockSpec((BT, KVLR), lambda i: (i, 0)),
            pl.BlockSpec((BT, _ROPE), lambda i: (i, 0)),
        ],
        out_shape=[
            jax.ShapeDtypeStruct((NH, S, KVLR), f16),
            jax.ShapeDtypeStruct((NH, S, _ROPE), f16),
            jax.ShapeDtypeStruct((S, KVLR), f16),
            jax.ShapeDtypeStruct((S, _ROPE), f16),
        ],
        compiler_params=pltpu.CompilerParams(
            dimension_semantics=("arbitrary",)),
    )
    q2, qpe, kv, kpe = k1(hid, cos, sin, ln1, wqaT, qnw, wqbT, wkvaT, kvnw, wkvbn)

    k2 = pl.pallas_call(
        functools.partial(_k2_body, scale),
        grid=(NH, NB),
        in_specs=[
            pl.BlockSpec((1, BT, KVLR), lambda h, i: (h, i, 0)),
            pl.BlockSpec((1, BT, _ROPE), lambda h, i: (h, i, 0)),
            pl.BlockSpec((S, KVLR), lambda h, i: (0, 0)),
            pl.BlockSpec((S, _ROPE), lambda h, i: (0, 0)),
        ],
        out_specs=pl.BlockSpec((1, BT, KVLR), lambda h, i: (h, i, 0)),
        out_shape=jax.ShapeDtypeStruct((NH, S, KVLR), f16),
        compiler_params=pltpu.CompilerParams(
            dimension_semantics=("arbitrary", "arbitrary")),
    )
    o = k2(q2, qpe, kv, kpe)

    k3 = pl.pallas_call(
        functools.partial(_k3_body, NH),
        grid=(NB,),
        in_specs=[
            pl.BlockSpec((NH, BT, KVLR), lambda i: (0, i, 0)),
            pl.BlockSpec((NH, KVLR, _VHD), lambda i: (0, 0, 0)),
            pl.BlockSpec((NH * _VHD, HS), lambda i: (0, 0)),
            pl.BlockSpec((1, HS), lambda i: (0, 0)),
            pl.BlockSpec((BT, HS), lambda i: (i, 0)),
            pl.BlockSpec((1, HS), lambda i: (0, 0)),
            pl.BlockSpec((HS, EN), lambda i: (0, 0)),
            pl.BlockSpec((1, EN), lambda i: (0, 0)),
        ],
        out_specs=[
            pl.BlockSpec((BT, HS), lambda i: (i, 0)),
            pl.BlockSpec((BT, HS), lambda i: (i, 0)),
            pl.BlockSpec((BT, EN), lambda i: (i, 0)),
            pl.BlockSpec((BT, 1), lambda i: (i, 0)),
            pl.BlockSpec((BT, 1), lambda i: (i, 0)),
            pl.BlockSpec((BT, 1), lambda i: (i, 0)),
            pl.BlockSpec((BT, 1), lambda i: (i, 0)),
        ],
        out_shape=[
            jax.ShapeDtypeStruct((S, HS), jnp.float32),
            jax.ShapeDtypeStruct((S, HS), jnp.float32),
            jax.ShapeDtypeStruct((S, EN), jnp.float32),
            jax.ShapeDtypeStruct((S, 1), jnp.int32),
            jax.ShapeDtypeStruct((S, 1), jnp.int32),
            jax.ShapeDtypeStruct((S, 1), jnp.float32),
            jax.ShapeDtypeStruct((S, 1), jnp.float32),
        ],
        compiler_params=pltpu.CompilerParams(
            dimension_semantics=("arbitrary",)),
    )
    h2, y, logits, i1, i2, w1, w2 = k3(o, wv, woT, wob, hid, ln2, gT, gb)

    BLKM = min(256, S)
    NPB = (2 * S) // BLKM + EN
    k4 = pl.pallas_call(
        functools.partial(_k4_body, EN, BLKM, NPB),
        grid=(1,),
        in_specs=[
            pl.BlockSpec((S, 1), lambda i: (0, 0)),
            pl.BlockSpec((S, 1), lambda i: (0, 0)),
        ],
        out_specs=[
            pl.BlockSpec((S, 1), lambda i: (0, 0)),
            pl.BlockSpec((S, 1), lambda i: (0, 0)),
            pl.BlockSpec((NPB, BLKM), lambda i: (0, 0)),
            pl.BlockSpec((NPB, 1), lambda i: (0, 0)),
        ],
        out_shape=[
            jax.ShapeDtypeStruct((S, 1), jnp.int32),
            jax.ShapeDtypeStruct((S, 1), jnp.int32),
            jax.ShapeDtypeStruct((NPB, BLKM), jnp.int32),
            jax.ShapeDtypeStruct((NPB, 1), jnp.int32),
        ],
    )
    pos1, pos2, idx_sorted, blk_exp = k4(i1, i2)

    idx_flat = idx_sorted.reshape(NPB * BLKM)
    be_flat = blk_exp.reshape(NPB)
    pos_flat = jnp.concatenate([pos1.reshape(S), pos2.reshape(S)])

    ys = _sc_gather(y, idx_flat)

    k6 = pl.pallas_call(
        _k6_body,
        grid_spec=pltpu.PrefetchScalarGridSpec(
            num_scalar_prefetch=1,
            grid=(NPB,),
            in_specs=[
                pl.BlockSpec((BLKM, HS), lambda b, bes: (b, 0)),
                pl.BlockSpec((1, HS, EK), lambda b, bes: (bes[b], 0, 0)),
                pl.BlockSpec((1, HS, EK), lambda b, bes: (bes[b], 0, 0)),
                pl.BlockSpec((1, EK, HS), lambda b, bes: (bes[b], 0, 0)),
            ],
            out_specs=pl.BlockSpec((BLKM, HS), lambda b, bes: (b, 0)),
        ),
        out_shape=jax.ShapeDtypeStruct((NPB * BLKM, HS), jnp.float32),
        compiler_params=pltpu.CompilerParams(
            dimension_semantics=("arbitrary",)),
    )
    eo = k6(be_flat, ys, egT, euT, edT)

    go = _sc_gather(eo, pos_flat)

    k7 = pl.pallas_call(
        _k7_body,
        grid=(NB,),
        in_specs=[
            pl.BlockSpec((BT, HS), lambda i: (i, 0)),
            pl.BlockSpec((BT, HS), lambda i: (NB + i, 0)),
            pl.BlockSpec((BT, HS), lambda i: (i, 0)),
            pl.BlockSpec((BT, 1), lambda i: (i, 0)),
            pl.BlockSpec((BT, 1), lambda i: (i, 0)),
        ],
        out_specs=pl.BlockSpec((BT, HS), lambda i: (i, 0)),
        out_shape=jax.ShapeDtypeStruct((S, HS), jnp.float32),
        compiler_params=pltpu.CompilerParams(
            dimension_semantics=("arbitrary",)),
    )
    out = k7(go, go, h2, w1, w2)

    return out.reshape(bs, S, HS), logits


# speedup vs baseline: 1.4404x; 1.0039x over previous
import functools
import math

import jax
import jax.numpy as jnp
from jax.experimental import pallas as pl
from jax.experimental.pallas import tpu as pltpu
from jax.experimental.pallas import tpu_sc as plsc

_NOPE = 128
_ROPE = 64
_VHD = 128
_EPS = 1e-6


def _rms(x, w):
    var = jnp.mean(x * x, axis=-1, keepdims=True)
    return (x * jax.lax.rsqrt(var + _EPS)) * w


def _rot_half(x):
    half = x.shape[-1] // 2
    return jnp.concatenate([-x[:, half:], x[:, :half]], axis=-1)


def _nt_dot(a, b):
    return jax.lax.dot_general(a, b, (((1,), (1,)), ((), ())),
                               preferred_element_type=jnp.float32)


def _k1_body(nh, nope, rope, hid_ref, cos_ref, sin_ref, ln1_ref, wqa_ref, qnw_ref,
             wqb_ref, wkva_ref, kvnw_ref, wkvbn_ref,
             q2_ref, qpe_ref, kv_ref, kpe_ref):
    x = hid_ref[...]
    xb = _rms(x, ln1_ref[...]).astype(jnp.bfloat16)
    qa = jnp.dot(xb, wqa_ref[...], preferred_element_type=jnp.float32)
    qab = _rms(qa, qnw_ref[...]).astype(jnp.bfloat16)
    q = jnp.dot(qab, wqb_ref[...], preferred_element_type=jnp.float32)
    kvf = jnp.dot(xb, wkva_ref[...], preferred_element_type=jnp.float32)
    kvlr = kvnw_ref.shape[-1]
    kv_ref[...] = _rms(kvf[:, :kvlr], kvnw_ref[...]).astype(jnp.bfloat16)
    cos = cos_ref[...]
    sin = sin_ref[...]
    kpe = kvf[:, kvlr:]
    kpe_ref[...] = (kpe * cos + _rot_half(kpe) * sin).astype(jnp.bfloat16)
    qkhd = nope + rope
    for h in range(nh):
        qn = q[:, h * qkhd:h * qkhd + nope].astype(jnp.bfloat16)
        q2_ref[h] = jnp.dot(qn, wkvbn_ref[h],
                            preferred_element_type=jnp.float32).astype(jnp.bfloat16)
        qp = q[:, h * qkhd + nope:(h + 1) * qkhd]
        qpe_ref[h] = (qp * cos + _rot_half(qp) * sin).astype(jnp.bfloat16)


def _k2_body(scale, q2_ref, qpe_ref, kv_ref, kpe_ref, o_ref):
    qb = pl.program_id(1)
    bt = q2_ref.shape[1]
    s_all = kv_ref.shape[0]
    q2 = q2_ref[0]
    qpe = qpe_ref[0]
    s = _nt_dot(q2, kv_ref[...]) + _nt_dot(qpe, kpe_ref[...])
    s = s * scale
    rows = qb * bt + jax.lax.broadcasted_iota(jnp.int32, (bt, s_all), 0)
    cols = jax.lax.broadcasted_iota(jnp.int32, (bt, s_all), 1)
    s = jnp.where(cols > rows, -1e30, s)
    m = jnp.max(s, axis=-1, keepdims=True)
    p = jnp.exp(s - m)
    l = jnp.sum(p, axis=-1, keepdims=True)
    acc = jnp.dot(p.astype(jnp.bfloat16), kv_ref[...],
                  preferred_element_type=jnp.float32)
    o_ref[0] = (acc / l).astype(jnp.bfloat16)


def _k3_body(nh, o_ref, wv_ref, woT_ref, wob_ref, hid_ref, ln2_ref, gT_ref, gb_ref,
             h2_ref, y_ref, lg_ref, i1_ref, i2_ref, w1_ref, w2_ref):
    parts = [jnp.dot(o_ref[h], wv_ref[h], preferred_element_type=jnp.float32)
             for h in range(nh)]
    o2 = jnp.concatenate(parts, axis=-1).astype(jnp.bfloat16)
    attn = jnp.dot(o2, woT_ref[...], preferred_element_type=jnp.float32) + wob_ref[...]
    h2 = hid_ref[...] + attn
    h2_ref[...] = h2
    y = _rms(h2, ln2_ref[...])
    y_ref[...] = y
    yb = y.astype(jnp.bfloat16)
    lg = jnp.dot(yb, gT_ref[...], preferred_element_type=jnp.float32) + gb_ref[...]
    lg_ref[...] = lg
    en = lg.shape[-1]
    col = jax.lax.broadcasted_iota(jnp.int32, lg.shape, 1)
    m1 = jnp.max(lg, axis=-1, keepdims=True)
    i1 = jnp.min(jnp.where(lg == m1, col, en), axis=-1, keepdims=True)
    l2 = jnp.where(col == i1, -jnp.inf, lg)
    m2 = jnp.max(l2, axis=-1, keepdims=True)
    i2 = jnp.min(jnp.where(l2 == m2, col, en), axis=-1, keepdims=True)
    i1_ref[...] = i1
    i2_ref[...] = i2
    e2 = jnp.exp(m2 - m1)
    denom = 1.0 + e2
    w1_ref[...] = 1.0 / denom
    w2_ref[...] = e2 / denom


def _k4_body(en, blk, npb, i1_ref, i2_ref, pos1_ref, pos2_ref, idx_ref, be_ref):
    i1 = i1_ref[...]
    i2 = i2_ref[...]
    s = i1.shape[0]
    eio = jax.lax.broadcasted_iota(jnp.int32, (s, en), 1)
    m1 = (i1 == eio).astype(jnp.float32)
    m2 = (i2 == eio).astype(jnp.float32)
    cnt1 = jnp.sum(m1, axis=0, keepdims=True)
    cnt2 = jnp.sum(m2, axis=0, keepdims=True)
    cnt = cnt1 + cnt2
    pc = jnp.floor((cnt + (blk - 1)) / blk) * blk
    r8 = jax.lax.broadcasted_iota(jnp.int32, (en, en), 0)
    c8 = jax.lax.broadcasted_iota(jnp.int32, (en, en), 1)
    tri = (r8 < c8).astype(jnp.float32)
    off = jnp.dot(pc, tri, preferred_element_type=jnp.float32)
    nc = s // blk
    p1c, p2c = [], []
    for c in range(nc):
        rowg = c * blk + jax.lax.broadcasted_iota(jnp.int32, (blk, s), 0)
        colg = jax.lax.broadcasted_iota(jnp.int32, (blk, s), 1)
        lt = (colg < rowg).astype(jnp.float32)
        p1c.append(jnp.dot(lt, m1, preferred_element_type=jnp.float32))
        p2c.append(jnp.dot(lt, m2, preferred_element_type=jnp.float32))
    pref1 = jnp.concatenate(p1c, axis=0)
    pref2 = jnp.concatenate(p2c, axis=0)
    rank1 = jnp.sum(pref1 * m1, axis=1, keepdims=True)
    rank2 = jnp.sum(pref2 * m2, axis=1, keepdims=True)
    off1 = jnp.sum(off * m1, axis=1, keepdims=True)
    base2 = off + cnt1
    off2 = jnp.sum(base2 * m2, axis=1, keepdims=True)
    pos1 = off1 + rank1
    pos2 = off2 + rank2
    pos1_ref[...] = pos1.astype(jnp.int32)
    pos2_ref[...] = pos2.astype(jnp.int32)
    tok = jax.lax.broadcasted_iota(jnp.int32, (s, blk), 0)
    p1i = pos1.astype(jnp.int32)
    p2i = pos2.astype(jnp.int32)
    rows = []
    for c in range(npb):
        slot = c * blk + jax.lax.broadcasted_iota(jnp.int32, (s, blk), 1)
        hit1 = jnp.where(p1i == slot, tok, -1)
        hit2 = jnp.where(p2i == slot, tok, -1)
        row = jnp.maximum(jnp.max(hit1, axis=0, keepdims=True),
                          jnp.max(hit2, axis=0, keepdims=True))
        rows.append(row)
    idx_ref[...] = jnp.maximum(jnp.concatenate(rows, axis=0), 0)
    bstart = jax.lax.broadcasted_iota(jnp.int32, (npb, en), 0) * blk
    offi = off.astype(jnp.int32)
    be = jnp.sum((bstart >= offi).astype(jnp.int32), axis=1, keepdims=True) - 1
    be_ref[...] = jnp.clip(be, 0, en - 1)


def _k6_body(be_ref, ys_ref, eg_ref, eu_ref, edT_ref, eo_ref):
    x = ys_ref[...].astype(jnp.bfloat16)
    g = jnp.dot(x, eg_ref[0], preferred_element_type=jnp.float32)
    u = jnp.dot(x, eu_ref[0], preferred_element_type=jnp.float32)
    act = (g * jax.nn.sigmoid(g) * u).astype(jnp.bfloat16)
    eo_ref[...] = jnp.dot(act, edT_ref[0], preferred_element_type=jnp.float32)


def _k7_body(g1_ref, g2_ref, h2_ref, w1_ref, w2_ref, out_ref):
    out_ref[...] = (h2_ref[...]
                    + w1_ref[...] * g1_ref[...]
                    + w2_ref[...] * g2_ref[...])


def _sc_gather(table, idx):
    info = plsc.get_sparse_core_info()
    nw = info.num_cores * info.num_subcores
    b = idx.shape[0]
    v, d = table.shape
    b_per_w = b // nw
    ch = next(c for c in (32, 24, 16, 8) if b_per_w % c == 0)
    n_ch = b_per_w // ch
    nc = info.num_cores
    row_shape = (ch, d)
    out_sds = jax.ShapeDtypeStruct((b, d), table.dtype)
    mesh = plsc.VectorSubcoreMesh(core_axis_name="c", subcore_axis_name="s")

    @functools.partial(
        pl.kernel, mesh=mesh,
        out_type=out_sds,
        scratch_types=[
            pltpu.VMEM((ch,), jnp.int32),
            pltpu.VMEM(row_shape, table.dtype),
            pltpu.SemaphoreType.DMA,
        ],
    )
    def k(table_hbm, idx_hbm, out_hbm, idx_v, rows_v, sem):
        wid = jax.lax.axis_index("s") * nc + jax.lax.axis_index("c")
        base = wid * b_per_w

        def step(j, carry):
            off = base + j * ch
            pltpu.sync_copy(idx_hbm.at[pl.ds(off, ch)], idx_v)
            pltpu.async_copy(table_hbm.at[idx_v], rows_v, sem).wait()
            pltpu.sync_copy(rows_v, out_hbm.at[pl.ds(off, ch)])
            return carry

        jax.lax.fori_loop(0, n_ch, step, 0)

    return k(table, idx)


def kernel(hidden_state, attention_mask, ln1_w, ln2_w, wq_a_w, wq_a_b, q_norm_w,
           wq_b_w, wq_b_b, wkv_a_w, wkv_a_b, kv_norm_w, wkv_b_w, wo_w, wo_b,
           gate_w, gate_b, eg_w, eu_w, ed_w):
    bs, S, HS = hidden_state.shape
    hid = hidden_state.reshape(S, HS)
    QLR = wq_a_w.shape[0]
    NH = wq_b_w.shape[0] // (_NOPE + _ROPE)
    KVLR = kv_norm_w.shape[0]
    EN, EK, _ = eg_w.shape
    qkhd = _NOPE + _ROPE
    scale = 1.0 / math.sqrt(float(qkhd))

    inv_freq = 1.0 / (10000.0 ** (jnp.arange(0, _ROPE, 2, dtype=jnp.float32) / _ROPE))
    t = jnp.arange(S, dtype=jnp.float32)[:, None]
    freqs = t * inv_freq[None, :]
    freqs = jnp.concatenate([freqs, freqs], axis=-1)
    cos = jnp.cos(freqs)
    sin = jnp.sin(freqs)

    f16 = jnp.bfloat16
    wqaT = wq_a_w.T.astype(f16)
    wqbT = wq_b_w.T.astype(f16)
    wkvaT = wkv_a_w.T.astype(f16)
    wkvb = wkv_b_w.reshape(NH, _NOPE + _VHD, KVLR)
    wkvbn = wkvb[:, :_NOPE, :].astype(f16)
    wv = wkvb[:, _NOPE:, :].transpose(0, 2, 1).astype(f16)
    woT = wo_w.T.astype(f16)
    gT = gate_w.T.astype(f16)
    egT = eg_w.transpose(0, 2, 1).astype(f16)
    euT = eu_w.transpose(0, 2, 1).astype(f16)
    edT = ed_w.transpose(0, 2, 1).astype(f16)
    ln1 = ln1_w.reshape(1, HS)
    ln2 = ln2_w.reshape(1, HS)
    qnw = q_norm_w.reshape(1, QLR)
    kvnw = kv_norm_w.reshape(1, KVLR)
    wob = wo_b.reshape(1, HS)
    gb = gate_b.reshape(1, EN)

    BT = min(256, S)
    NB = S // BT

    k1 = pl.pallas_call(
        functools.partial(_k1_body, NH, _NOPE, _ROPE),
        grid=(NB,),
        in_specs=[
            pl.BlockSpec((BT, HS), lambda i: (i, 0)),
            pl.BlockSpec((BT, _ROPE), lambda i: (i, 0)),
            pl.BlockSpec((BT, _ROPE), lambda i: (i, 0)),
            pl.BlockSpec((1, HS), lambda i: (0, 0)),
            pl.BlockSpec((HS, QLR), lambda i: (0, 0)),
            pl.BlockSpec((1, QLR), lambda i: (0, 0)),
            pl.BlockSpec((QLR, NH * qkhd), lambda i: (0, 0)),
            pl.BlockSpec((HS, KVLR + _ROPE), lambda i: (0, 0)),
            pl.BlockSpec((1, KVLR), lambda i: (0, 0)),
            pl.BlockSpec((NH, _NOPE, KVLR), lambda i: (0, 0, 0)),
        ],
        out_specs=[
            pl.BlockSpec((NH, BT, KVLR), lambda i: (0, i, 0)),
            pl.BlockSpec((NH, BT, _ROPE), lambda i: (0, i, 0)),
            pl.BlockSpec((BT, KVLR), lambda i: (i, 0)),
            pl.BlockSpec((BT, _ROPE), lambda i: (i, 0)),
        ],
        out_shape=[
            jax.ShapeDtypeStruct((NH, S, KVLR), f16),
            jax.ShapeDtypeStruct((NH, S, _ROPE), f16),
            jax.ShapeDtypeStruct((S, KVLR), f16),
            jax.ShapeDtypeStruct((S, _ROPE), f16),
        ],
        compiler_params=pltpu.CompilerParams(
            dimension_semantics=("arbitrary",)),
    )
    q2, qpe, kv, kpe = k1(hid, cos, sin, ln1, wqaT, qnw, wqbT, wkvaT, kvnw, wkvbn)

    k2 = pl.pallas_call(
        functools.partial(_k2_body, scale),
        grid=(NH, NB),
        in_specs=[
            pl.BlockSpec((1, BT, KVLR), lambda h, i: (h, i, 0)),
            pl.BlockSpec((1, BT, _ROPE), lambda h, i: (h, i, 0)),
            pl.BlockSpec((S, KVLR), lambda h, i: (0, 0)),
            pl.BlockSpec((S, _ROPE), lambda h, i: (0, 0)),
        ],
        out_specs=pl.BlockSpec((1, BT, KVLR), lambda h, i: (h, i, 0)),
        out_shape=jax.ShapeDtypeStruct((NH, S, KVLR), f16),
        compiler_params=pltpu.CompilerParams(
            dimension_semantics=("arbitrary", "arbitrary")),
    )
    o = k2(q2, qpe, kv, kpe)

    k3 = pl.pallas_call(
        functools.partial(_k3_body, NH),
        grid=(NB,),
        in_specs=[
            pl.BlockSpec((NH, BT, KVLR), lambda i: (0, i, 0)),
            pl.BlockSpec((NH, KVLR, _VHD), lambda i: (0, 0, 0)),
            pl.BlockSpec((NH * _VHD, HS), lambda i: (0, 0)),
            pl.BlockSpec((1, HS), lambda i: (0, 0)),
            pl.BlockSpec((BT, HS), lambda i: (i, 0)),
            pl.BlockSpec((1, HS), lambda i: (0, 0)),
            pl.BlockSpec((HS, EN), lambda i: (0, 0)),
            pl.BlockSpec((1, EN), lambda i: (0, 0)),
        ],
        out_specs=[
            pl.BlockSpec((BT, HS), lambda i: (i, 0)),
            pl.BlockSpec((BT, HS), lambda i: (i, 0)),
            pl.BlockSpec((BT, EN), lambda i: (i, 0)),
            pl.BlockSpec((BT, 1), lambda i: (i, 0)),
            pl.BlockSpec((BT, 1), lambda i: (i, 0)),
            pl.BlockSpec((BT, 1), lambda i: (i, 0)),
            pl.BlockSpec((BT, 1), lambda i: (i, 0)),
        ],
        out_shape=[
            jax.ShapeDtypeStruct((S, HS), jnp.float32),
            jax.ShapeDtypeStruct((S, HS), jnp.float32),
            jax.ShapeDtypeStruct((S, EN), jnp.float32),
            jax.ShapeDtypeStruct((S, 1), jnp.int32),
            jax.ShapeDtypeStruct((S, 1), jnp.int32),
            jax.ShapeDtypeStruct((S, 1), jnp.float32),
            jax.ShapeDtypeStruct((S, 1), jnp.float32),
        ],
        compiler_params=pltpu.CompilerParams(
            dimension_semantics=("arbitrary",)),
    )
    h2, y, logits, i1, i2, w1, w2 = k3(o, wv, woT, wob, hid, ln2, gT, gb)

    BLKM = min(256, S)
    NPB = (2 * S) // BLKM + EN
    k4 = pl.pallas_call(
        functools.partial(_k4_body, EN, BLKM, NPB),
        grid=(1,),
        in_specs=[
            pl.BlockSpec((S, 1), lambda i: (0, 0)),
            pl.BlockSpec((S, 1), lambda i: (0, 0)),
        ],
        out_specs=[
            pl.BlockSpec((S, 1), lambda i: (0, 0)),
            pl.BlockSpec((S, 1), lambda i: (0, 0)),
            pl.BlockSpec((NPB, BLKM), lambda i: (0, 0)),
            pl.BlockSpec((NPB, 1), lambda i: (0, 0)),
        ],
        out_shape=[
            jax.ShapeDtypeStruct((S, 1), jnp.int32),
            jax.ShapeDtypeStruct((S, 1), jnp.int32),
            jax.ShapeDtypeStruct((NPB, BLKM), jnp.int32),
            jax.ShapeDtypeStruct((NPB, 1), jnp.int32),
        ],
    )
    pos1, pos2, idx_sorted, blk_exp = k4(i1, i2)

    idx_flat = idx_sorted.reshape(NPB * BLKM)
    be_flat = blk_exp.reshape(NPB)
    pos_flat = jnp.concatenate([pos1.reshape(S), pos2.reshape(S)])

    ys = _sc_gather(y, idx_flat)

    k6 = pl.pallas_call(
        _k6_body,
        grid_spec=pltpu.PrefetchScalarGridSpec(
            num_scalar_prefetch=1,
            grid=(NPB,),
            in_specs=[
                pl.BlockSpec((BLKM, HS), lambda b, bes: (b, 0)),
                pl.BlockSpec((1, HS, EK), lambda b, bes: (bes[b], 0, 0)),
                pl.BlockSpec((1, HS, EK), lambda b, bes: (bes[b], 0, 0)),
                pl.BlockSpec((1, EK, HS), lambda b, bes: (bes[b], 0, 0)),
            ],
            out_specs=pl.BlockSpec((BLKM, HS), lambda b, bes: (b, 0)),
        ),
        out_shape=jax.ShapeDtypeStruct((NPB * BLKM, HS), jnp.float32),
        compiler_params=pltpu.CompilerParams(
            dimension_semantics=("arbitrary",)),
    )
    eo = k6(be_flat, ys, egT, euT, edT)

    go = _sc_gather(eo, pos_flat)

    k7 = pl.pallas_call(
        _k7_body,
        grid=(NB,),
        in_specs=[
            pl.BlockSpec((BT, HS), lambda i: (i, 0)),
            pl.BlockSpec((BT, HS), lambda i: (NB + i, 0)),
            pl.BlockSpec((BT, HS), lambda i: (i, 0)),
            pl.BlockSpec((BT, 1), lambda i: (i, 0)),
            pl.BlockSpec((BT, 1), lambda i: (i, 0)),
        ],
        out_specs=pl.BlockSpec((BT, HS), lambda i: (i, 0)),
        out_shape=jax.ShapeDtypeStruct((S, HS), jnp.float32),
        compiler_params=pltpu.CompilerParams(
            dimension_semantics=("arbitrary",)),
    )
    out = k7(go, go, h2, w1, w2)

    return out.reshape(bs, S, HS), logits
